# baseline (speedup 1.0000x reference)
import functools

import jax
import jax.numpy as jnp
from jax import lax
from jax.experimental import pallas as pl
from jax.experimental.pallas import tpu as pltpu
from jax.experimental.pallas import tpu_sc as plsc

B = 4096
L = 200
DIM = 64
PD = 128
NW = 32
CH = 128
CPG = 2
G = CH * CPG
TOTAL = B * L
B_PER_W = TOTAL // NW
N_CHUNKS = B_PER_W // CH
NG = B_PER_W // G


def _gather(xf, table_p):
    mesh = plsc.VectorSubcoreMesh(core_axis_name="c", subcore_axis_name="s")
    nc = 2

    @functools.partial(
        pl.kernel,
        out_type=jax.ShapeDtypeStruct((NW, B_PER_W, PD), jnp.float32),
        mesh=mesh,
        scratch_types=[
            pltpu.VMEM((N_CHUNKS, CH), jnp.int32),
            pltpu.VMEM((2, G, PD), jnp.float32),
            pltpu.SemaphoreType.DMA,
            pltpu.SemaphoreType.DMA,
        ],
    )
    def k(x_hbm, table_hbm, out_hbm, idx_v, rows_v, gsem, wsem):
        wid = lax.axis_index("s") * nc + lax.axis_index("c")
        pltpu.sync_copy(x_hbm.at[wid], idx_v)

        def fire_gathers(g, p):
            for c in range(CPG):
                pltpu.async_copy(
                    table_hbm.at[idx_v.at[g * CPG + c]],
                    rows_v.at[p, pl.ds(c * CH, CH)],
                    gsem,
                )

        def drain_gathers(p):
            for c in range(CPG):
                pltpu.make_async_copy(
                    table_hbm.at[idx_v.at[c]],
                    rows_v.at[p, pl.ds(c * CH, CH)],
                    gsem,
                ).wait()

        def fire_wb(g, p):
            pltpu.async_copy(
                rows_v.at[p],
                out_hbm.at[wid, pl.ds(g * G, G)],
                wsem,
            )

        def drain_wb(g, p):
            pltpu.make_async_copy(
                rows_v.at[p],
                out_hbm.at[wid, pl.ds(g * G, G)],
                wsem,
            ).wait()

        fire_gathers(0, 0)

        def outer(Gi, _):
            for p in range(2):
                g = Gi * 2 + p
                drain_gathers(p)
                fire_wb(g, p)
                @pl.when(g > 0)
                def _():
                    drain_wb(g - 1, 1 - p)
                @pl.when(g + 1 < NG)
                def _():
                    fire_gathers(g + 1, 1 - p)
            return 0

        lax.fori_loop(0, NG // 2, outer, 0, unroll=False)
        drain_wb(NG - 1, 1)

    return k(xf, table_p)


def kernel(x, table):
    table_p = table.reshape(table.shape[0] // 2, PD)
    xf = x.reshape(NW, N_CHUNKS, CH)
    out = _gather(xf >> 1, table_p)
    par = (x.reshape(NW, B_PER_W, 1) & 1).astype(jnp.bool_)
    out64 = jnp.where(par, out[:, :, DIM:], out[:, :, :DIM])
    return out64.reshape(B, L, DIM)
